# baseline (speedup 1.0000x reference)
import functools

import jax
import jax.numpy as jnp
from jax import lax
from jax.experimental import pallas as pl
from jax.experimental.pallas import tpu as pltpu
from jax.experimental.pallas import tpu_sc as plsc

NC = 2
NS = 16
NW = NC * NS
LANES = 16
MAGIC = 0x5F3759DF


def _pick_block(ept: int, maxb: int, even_count: bool = False) -> int:
    align = 8 if even_count else 16
    for b in range(min(maxb, ept), 15, -1):
        if ept % b == 0 and b % align == 0:
            if even_count and (ept // b) % 2 != 0:
                continue
            return b
    return 0


def _iota16():
    return lax.broadcasted_iota(jnp.int32, (LANES,), 0)


def _make_deg(n: int, e: int):
    ept = e // NW
    b1 = _pick_block(ept, 10000)
    e2 = e // NC
    mesh = plsc.VectorSubcoreMesh(core_axis_name="c", subcore_axis_name="s")

    @functools.partial(
        pl.kernel,
        out_type=jax.ShapeDtypeStruct((NC, n), jnp.float32),
        mesh=mesh,
        scratch_types=[
            pltpu.VMEM((b1,), jnp.int32),
            pltpu.VMEM((b1,), jnp.float32),
            pltpu.VMEM((b1,), jnp.float32),
            pltpu.VMEM_SHARED((n,), jnp.float32),
        ],
        compiler_params=pltpu.CompilerParams(use_tc_tiling_on_sc=False, needs_layout_passes=False),
    )
    def deg_kernel(ei_hbm, outp, idx_v, ones_v, zero_v, deg_sh):
        c = lax.axis_index("c")
        s = lax.axis_index("s")

        def fill(i, carry):
            ones_v[pl.ds(i * 16, 16)] = jnp.full((16,), 1.0, jnp.float32)
            zero_v[pl.ds(i * 16, 16)] = jnp.zeros((16,), jnp.float32)
            return carry

        lax.fori_loop(0, b1 // 16, fill, 0)

        @pl.when(s == 0)
        def _():
            def zchunk(i, carry):
                pltpu.sync_copy(zero_v, deg_sh.at[pl.ds(i * b1, b1)])
                return carry
            lax.fori_loop(0, n // b1, zchunk, 0)
            if n % b1:
                pltpu.sync_copy(zero_v.at[pl.ds(0, n % b1)],
                                deg_sh.at[pl.ds((n // b1) * b1, n % b1)])

        plsc.subcore_barrier()

        def step(i, carry):
            base = c * e2 + s * ept + i * b1
            pltpu.sync_copy(ei_hbm.at[1, pl.ds(base, b1)], idx_v)
            pltpu.sync_copy(ones_v, deg_sh.at[idx_v], add=True)
            return carry

        lax.fori_loop(0, ept // b1, step, 0)
        plsc.subcore_barrier()

        @pl.when(s == 0)
        def _():
            pltpu.sync_copy(deg_sh, outp.at[c])

    return deg_kernel


def _make_agg(n: int, e: int, f: int):
    ept = e // NW
    b3 = _pick_block(ept, 2000, even_count=True)
    e2 = e // NC
    nit = ept // b3
    chunk = 400
    big = ((n // NS + chunk - 1) // chunk) * chunk
    rest = n - big * (NS - 1)
    assert rest > 0 and rest % chunk == 0 and big % chunk == 0
    nrow = n * f // 128
    npad = ((n + 127) // 128) * 128
    rows_per_chunk = chunk * f // 128
    mesh = plsc.VectorSubcoreMesh(core_axis_name="c", subcore_axis_name="s")

    @functools.partial(
        pl.kernel,
        out_type=(
            jax.ShapeDtypeStruct((NC, nrow, 128), jnp.float32),
            jax.ShapeDtypeStruct((NC, npad), jnp.float32),
        ),
        mesh=mesh,
        scratch_types=[
            [pltpu.VMEM((b3,), jnp.int32)] * 2,
            [pltpu.VMEM((b3,), jnp.int32)] * 2,
            [pltpu.VMEM((b3, f), jnp.float32)] * 2,
            [pltpu.VMEM((b3,), jnp.float32)] * 2,
            pltpu.VMEM((chunk, f), jnp.float32),
            pltpu.VMEM((chunk, f), jnp.float32),
            pltpu.VMEM((chunk,), jnp.float32),
            pltpu.VMEM((chunk,), jnp.float32),
            pltpu.VMEM((chunk,), jnp.float32),
            pltpu.VMEM((chunk,), jnp.float32),
            pltpu.VMEM((chunk,), jnp.float32),
            pltpu.VMEM((rows_per_chunk, 128), jnp.float32),
            pltpu.VMEM((chunk, f), jnp.float32),
            pltpu.VMEM((chunk,), jnp.float32),
            pltpu.VMEM_SHARED((n, f), jnp.float32),
            pltpu.VMEM_SHARED((n,), jnp.float32),
            pltpu.VMEM_SHARED((n, f), jnp.float32),
            pltpu.VMEM_SHARED((n,), jnp.float32),
            [pltpu.SemaphoreType.DMA] * 2,
            [pltpu.SemaphoreType.DMA] * 2,
            [pltpu.SemaphoreType.DMA] * 2,
            [pltpu.SemaphoreType.DMA] * 2,
            [pltpu.SemaphoreType.DMA] * 2,
        ],
        compiler_params=pltpu.CompilerParams(use_tc_tiling_on_sc=False, needs_layout_passes=False),
    )
    def agg_kernel(ei_hbm, x_hbm, degp_hbm, a_out, w_out,
                   sidx_v, didx_v, rows_v, dval_v,
                   xbuf, ybuf, dbuf0, dbuf1, dvbuf, tbuf, wbuf, obuf,
                   zbuf8, zbuf1, y_sh, dinv_sh, agg_sh, t_sh,
                   isem, rsem, dsem, ssem, tsem):
        c = lax.axis_index("c")
        s = lax.axis_index("s")
        iota = _iota16()
        a0 = s * big
        nchunks = jnp.where(s == NS - 1, rest // chunk, big // chunk)

        zf32 = jnp.zeros((LANES,), jnp.float32)
        for v in range(chunk * f // LANES):
            fi = v * LANES + iota
            plsc.store_scatter(
                zbuf8, [lax.shift_right_logical(fi, 3),
                        lax.bitwise_and(fi, 7)], zf32)
        for v in range(chunk // LANES):
            zbuf1[pl.ds(v * LANES, LANES)] = zf32

        def phaseb(j, carry):
            a = a0 + j * chunk
            pltpu.sync_copy(degp_hbm.at[0, pl.ds(a, chunk)], dbuf0)
            pltpu.sync_copy(degp_hbm.at[1, pl.ds(a, chunk)], dbuf1)
            pltpu.sync_copy(x_hbm.at[pl.ds(a, chunk), :], xbuf)
            for v in range(chunk // LANES):
                sl = pl.ds(v * LANES, LANES)
                d0 = dbuf0[sl] + dbuf1[sl] + 1.0
                i0 = plsc.bitcast(d0, jnp.int32)
                i1 = MAGIC - lax.shift_right_arithmetic(i0, 1)
                r = plsc.bitcast(i1, jnp.float32)
                for _ in range(3):
                    r = r * (1.5 - 0.5 * d0 * r * r)
                dvbuf[sl] = r
            for v in range(chunk * f // LANES):
                fi = v * LANES + iota
                ri = lax.shift_right_logical(fi, 3)
                ci = lax.bitwise_and(fi, 7)
                xv = plsc.load_gather(xbuf, [ri, ci])
                dv = plsc.load_gather(dvbuf, [ri])
                plsc.store_scatter(ybuf, [ri, ci], xv * dv)
            pltpu.sync_copy(dvbuf, dinv_sh.at[pl.ds(a, chunk)])
            pltpu.sync_copy(ybuf, y_sh.at[pl.ds(a, chunk), :])

            @pl.when(c == 0)
            def _():
                pltpu.sync_copy(ybuf, agg_sh.at[pl.ds(a, chunk), :])
                pltpu.sync_copy(dvbuf, t_sh.at[pl.ds(a, chunk)])

            @pl.when(c != 0)
            def _():
                pltpu.sync_copy(zbuf8, agg_sh.at[pl.ds(a, chunk), :])
                pltpu.sync_copy(zbuf1, t_sh.at[pl.ds(a, chunk)])
            return carry

        lax.fori_loop(0, nchunks, phaseb, 0)
        plsc.subcore_barrier()

        def fetch_idx(i, k):
            base = c * e2 + s * ept + i * b3
            pltpu.async_copy(ei_hbm.at[0, pl.ds(base, b3)], sidx_v[k],
                             isem[k])
            pltpu.async_copy(ei_hbm.at[1, pl.ds(base, b3)], didx_v[k],
                             isem[k])

        def fetch_rows(k):
            pltpu.async_copy(y_sh.at[sidx_v[k]], rows_v[k], rsem[k])
            pltpu.async_copy(dinv_sh.at[didx_v[k]], dval_v[k], dsem[k])

        def wait_idx(i, k):
            base = c * e2 + s * ept + i * b3
            pltpu.make_async_copy(ei_hbm.at[0, pl.ds(base, b3)], sidx_v[k],
                                  isem[k]).wait()
            pltpu.make_async_copy(ei_hbm.at[1, pl.ds(base, b3)], didx_v[k],
                                  isem[k]).wait()

        fetch_idx(0, 0)
        wait_idx(0, 0)
        fetch_rows(0)
        fetch_idx(1, 1)

        def step(j, carry):
            for k in (0, 1):
                i = j * 2 + k
                kn = 1 - k
                pltpu.make_async_copy(y_sh.at[sidx_v[k]], rows_v[k],
                                      rsem[k]).wait()
                pltpu.make_async_copy(dinv_sh.at[didx_v[k]], dval_v[k],
                                      dsem[k]).wait()
                pltpu.async_copy(rows_v[k], agg_sh.at[didx_v[k]], ssem[k],
                                 add=True)
                pltpu.async_copy(dval_v[k], t_sh.at[sidx_v[k]], tsem[k],
                                 add=True)

                @pl.when(i + 1 < nit)
                def _():
                    wait_idx(i + 1, kn)
                    fetch_rows(kn)

                pltpu.make_async_copy(rows_v[k], agg_sh.at[didx_v[k]],
                                      ssem[k]).wait()
                pltpu.make_async_copy(dval_v[k], t_sh.at[sidx_v[k]],
                                      tsem[k]).wait()

                @pl.when(i + 2 < nit)
                def _():
                    fetch_idx(i + 2, k)
            return carry

        lax.fori_loop(0, nit // 2, step, 0)
        plsc.subcore_barrier()

        def phased(j, carry):
            a = a0 + j * chunk
            pltpu.sync_copy(agg_sh.at[pl.ds(a, chunk), :], xbuf)
            pltpu.sync_copy(dinv_sh.at[pl.ds(a, chunk)], dvbuf)
            pltpu.sync_copy(t_sh.at[pl.ds(a, chunk)], tbuf)
            for v in range(chunk * f // LANES):
                fi = v * LANES + iota
                ri = lax.shift_right_logical(fi, 3)
                ci = lax.bitwise_and(fi, 7)
                av = plsc.load_gather(xbuf, [ri, ci])
                dv = plsc.load_gather(dvbuf, [ri])
                ro = lax.shift_right_logical(fi, 7)
                co = lax.bitwise_and(fi, 127)
                plsc.store_scatter(obuf, [ro, co], av * dv)
            for v in range(chunk // LANES):
                sl = pl.ds(v * LANES, LANES)
                wbuf[sl] = dvbuf[sl] * tbuf[sl]
            pltpu.sync_copy(
                obuf, a_out.at[c, pl.ds(a * f // 128, rows_per_chunk), :])
            pltpu.sync_copy(wbuf, w_out.at[c, pl.ds(a, chunk)])
            return carry

        lax.fori_loop(0, nchunks, phased, 0)

        if npad > n:
            @pl.when(s == NS - 1)
            def _():
                pltpu.sync_copy(zbuf1.at[pl.ds(0, npad - n)],
                                w_out.at[c, pl.ds(n, npad - n)])

    return agg_kernel


def _make_dense(n: int, f: int):
    def dense_body(a_in, wpt, mask, fsel, w1bd, b1t, st,
                   w2r, b2r, wsr, bsr, wor, bor, out):
        a = a_in[0] + a_in[1]
        p = jnp.dot(a, w1bd[...],
                    preferred_element_type=jnp.float32)
        h = jnp.maximum(p + b1t[...], 0.0)
        cm = jnp.dot(wpt[...], h,
                     preferred_element_type=jnp.float32)
        colsum = jnp.sum(cm * mask[...], axis=0, keepdims=True)
        colsum = jnp.dot(colsum, fsel[...],
                         preferred_element_type=jnp.float32)
        gm = jnp.dot(colsum * (1.0 / n), w2r[...],
                     preferred_element_type=jnp.float32) + b2r[...]
        se = jnp.maximum(
            jnp.dot(st[...], wsr[...],
                    preferred_element_type=jnp.float32) + bsr[...], 0.0)
        z = jnp.concatenate([gm, se], axis=1)
        out[...] = jnp.dot(z, wor[...],
                           preferred_element_type=jnp.float32) + bor[...]

    return pl.pallas_call(
        dense_body,
        out_shape=jax.ShapeDtypeStruct((1, 2), jnp.float32),
    )


def kernel(x, edge_index, state, W1, b1, W2, b2, Ws, bs, Wo, bo):
    n, f = x.shape
    e = edge_index.shape[1]
    g = 128 // f
    npad = ((n + 127) // 128) * 128

    degp = _make_deg(n, e)(edge_index)
    a_out, w_out = _make_agg(n, e, f)(edge_index, x, degp)

    w1bd = (jnp.eye(g, dtype=jnp.float32)[:, None, :, None]
            * W1[None, :, None, :]).reshape(f * g, 64 * g)
    b1t = jnp.tile(b1, g)[None]

    nrow = n * f // 128
    wsum = (w_out[0] + w_out[1])[:n]
    wpt = wsum.reshape(nrow, g).T
    mask = jnp.repeat(jnp.eye(g, dtype=jnp.float32), 64, axis=1)
    fsel = jnp.tile(jnp.eye(64, dtype=jnp.float32), (g, 1))

    out = _make_dense(n, f)(
        a_out, wpt, mask, fsel, w1bd, b1t, state,
        W2, b2[None], Ws, bs[None], Wo, bo[None])
    return out


# speedup vs baseline: 1.1202x; 1.1202x over previous
import functools

import jax
import jax.numpy as jnp
from jax import lax
from jax.experimental import pallas as pl
from jax.experimental.pallas import tpu as pltpu
from jax.experimental.pallas import tpu_sc as plsc

NC = 2
NS = 16
NW = NC * NS
LANES = 16
MAGIC = 0x5F3759DF


def _pick_block(ept: int, maxb: int, even_count: bool = False) -> int:
    align = 8 if even_count else 16
    for b in range(min(maxb, ept), 15, -1):
        if ept % b == 0 and b % align == 0:
            if even_count and (ept // b) % 2 != 0:
                continue
            return b
    return 0


def _iota16():
    return lax.broadcasted_iota(jnp.int32, (LANES,), 0)


def _make_deg(n: int, e: int):
    ept = e // NW
    b1 = _pick_block(ept, 10000)
    e2 = e // NC
    mesh = plsc.VectorSubcoreMesh(core_axis_name="c", subcore_axis_name="s")

    @functools.partial(
        pl.kernel,
        out_type=jax.ShapeDtypeStruct((NC, n), jnp.float32),
        mesh=mesh,
        scratch_types=[
            [pltpu.VMEM((b1,), jnp.int32)] * 2,
            pltpu.VMEM((b1,), jnp.float32),
            pltpu.VMEM((b1,), jnp.float32),
            pltpu.VMEM_SHARED((n,), jnp.float32),
            [pltpu.SemaphoreType.DMA] * 2,
        ],
        compiler_params=pltpu.CompilerParams(use_tc_tiling_on_sc=False, needs_layout_passes=False),
    )
    def deg_kernel(ei_hbm, outp, idx_v, ones_v, zero_v, deg_sh, isem):
        c = lax.axis_index("c")
        s = lax.axis_index("s")

        nit = ept // b1

        def fetch(i):
            base = c * e2 + s * ept + i * b1
            pltpu.async_copy(ei_hbm.at[1, pl.ds(base, b1)], idx_v[i % 2],
                             isem[i % 2])

        def wait(i):
            base = c * e2 + s * ept + i * b1
            pltpu.make_async_copy(ei_hbm.at[1, pl.ds(base, b1)],
                                  idx_v[i % 2], isem[i % 2]).wait()

        fetch(0)

        def fill(i, carry):
            ones_v[pl.ds(i * 16, 16)] = jnp.full((16,), 1.0, jnp.float32)
            zero_v[pl.ds(i * 16, 16)] = jnp.zeros((16,), jnp.float32)
            return carry

        lax.fori_loop(0, b1 // 16, fill, 0)

        @pl.when(s == 0)
        def _():
            def zchunk(i, carry):
                pltpu.sync_copy(zero_v, deg_sh.at[pl.ds(i * b1, b1)])
                return carry
            lax.fori_loop(0, n // b1, zchunk, 0)
            if n % b1:
                pltpu.sync_copy(zero_v.at[pl.ds(0, n % b1)],
                                deg_sh.at[pl.ds((n // b1) * b1, n % b1)])

        plsc.subcore_barrier()

        for i in range(nit):
            wait(i)
            if i + 1 < nit:
                fetch(i + 1)
            pltpu.sync_copy(ones_v, deg_sh.at[idx_v[i % 2]], add=True)
        plsc.subcore_barrier()

        @pl.when(s == 0)
        def _():
            pltpu.sync_copy(deg_sh, outp.at[c])

    return deg_kernel


def _make_agg(n: int, e: int, f: int):
    ept = e // NW
    b3 = _pick_block(ept, 2000, even_count=True)
    e2 = e // NC
    nit = ept // b3
    chunk = 400
    big = ((n // NS + chunk - 1) // chunk) * chunk
    rest = n - big * (NS - 1)
    assert rest > 0 and rest % chunk == 0 and big % chunk == 0
    nrow = n * f // 128
    npad = ((n + 127) // 128) * 128
    rows_per_chunk = chunk * f // 128
    mesh = plsc.VectorSubcoreMesh(core_axis_name="c", subcore_axis_name="s")

    @functools.partial(
        pl.kernel,
        out_type=(
            jax.ShapeDtypeStruct((NC, nrow, 128), jnp.float32),
            jax.ShapeDtypeStruct((NC, npad), jnp.float32),
        ),
        mesh=mesh,
        scratch_types=[
            [pltpu.VMEM((b3,), jnp.int32)] * 2,
            [pltpu.VMEM((b3,), jnp.int32)] * 2,
            [pltpu.VMEM((b3, f), jnp.float32)] * 2,
            [pltpu.VMEM((b3,), jnp.float32)] * 2,
            pltpu.VMEM((chunk, f), jnp.float32),
            pltpu.VMEM((chunk, f), jnp.float32),
            pltpu.VMEM((chunk,), jnp.float32),
            pltpu.VMEM((chunk,), jnp.float32),
            pltpu.VMEM((chunk,), jnp.float32),
            pltpu.VMEM((chunk,), jnp.float32),
            pltpu.VMEM((chunk,), jnp.float32),
            pltpu.VMEM((rows_per_chunk, 128), jnp.float32),
            pltpu.VMEM((chunk, f), jnp.float32),
            pltpu.VMEM((chunk,), jnp.float32),
            pltpu.VMEM_SHARED((n, f), jnp.float32),
            pltpu.VMEM_SHARED((n,), jnp.float32),
            pltpu.VMEM_SHARED((n, f), jnp.float32),
            pltpu.VMEM_SHARED((n,), jnp.float32),
            [pltpu.SemaphoreType.DMA] * 2,
            [pltpu.SemaphoreType.DMA] * 2,
            [pltpu.SemaphoreType.DMA] * 2,
        ],
        compiler_params=pltpu.CompilerParams(use_tc_tiling_on_sc=False, needs_layout_passes=False),
    )
    def agg_kernel(ei_hbm, x_hbm, degp_hbm, a_out, w_out,
                   sidx_v, didx_v, rows_v, dval_v,
                   xbuf, ybuf, dbuf0, dbuf1, dvbuf, tbuf, wbuf, obuf,
                   zbuf8, zbuf1, y_sh, dinv_sh, agg_sh, t_sh,
                   isem, rsem, dsem):
        c = lax.axis_index("c")
        s = lax.axis_index("s")
        iota = _iota16()
        a0 = s * big
        nchunks = jnp.where(s == NS - 1, rest // chunk, big // chunk)

        zf32 = jnp.zeros((LANES,), jnp.float32)
        for v in range(chunk * f // LANES):
            fi = v * LANES + iota
            plsc.store_scatter(
                zbuf8, [lax.shift_right_logical(fi, 3),
                        lax.bitwise_and(fi, 7)], zf32)
        for v in range(chunk // LANES):
            zbuf1[pl.ds(v * LANES, LANES)] = zf32

        def phaseb(j, carry):
            a = a0 + j * chunk
            d0 = pltpu.async_copy(degp_hbm.at[0, pl.ds(a, chunk)], dbuf0,
                                  isem[0])
            d1 = pltpu.async_copy(degp_hbm.at[1, pl.ds(a, chunk)], dbuf1,
                                  isem[1])
            dx = pltpu.async_copy(x_hbm.at[pl.ds(a, chunk), :], xbuf,
                                  rsem[0])
            d0.wait()
            d1.wait()
            dx.wait()
            for v in range(chunk // LANES):
                sl = pl.ds(v * LANES, LANES)
                d0 = dbuf0[sl] + dbuf1[sl] + 1.0
                i0 = plsc.bitcast(d0, jnp.int32)
                i1 = MAGIC - lax.shift_right_arithmetic(i0, 1)
                r = plsc.bitcast(i1, jnp.float32)
                for _ in range(3):
                    r = r * (1.5 - 0.5 * d0 * r * r)
                dvbuf[sl] = r
            for v in range(chunk * f // LANES):
                fi = v * LANES + iota
                ri = lax.shift_right_logical(fi, 3)
                ci = lax.bitwise_and(fi, 7)
                xv = plsc.load_gather(xbuf, [ri, ci])
                dv = plsc.load_gather(dvbuf, [ri])
                plsc.store_scatter(ybuf, [ri, ci], xv * dv)
            o0 = pltpu.async_copy(dvbuf, dinv_sh.at[pl.ds(a, chunk)],
                                  isem[0])
            o1 = pltpu.async_copy(ybuf, y_sh.at[pl.ds(a, chunk), :],
                                  isem[1])

            @pl.when(c == 0)
            def _():
                pltpu.async_copy(ybuf, agg_sh.at[pl.ds(a, chunk), :],
                                 rsem[0]).wait()
                pltpu.async_copy(dvbuf, t_sh.at[pl.ds(a, chunk)],
                                 rsem[1]).wait()

            @pl.when(c != 0)
            def _():
                pltpu.async_copy(zbuf8, agg_sh.at[pl.ds(a, chunk), :],
                                 rsem[0]).wait()
                pltpu.async_copy(zbuf1, t_sh.at[pl.ds(a, chunk)],
                                 rsem[1]).wait()
            o0.wait()
            o1.wait()
            return carry

        lax.fori_loop(0, nchunks, phaseb, 0)
        plsc.subcore_barrier()

        def fetch_idx(i, k):
            base = c * e2 + s * ept + i * b3
            pltpu.async_copy(ei_hbm.at[0, pl.ds(base, b3)], sidx_v[k],
                             isem[k])
            pltpu.async_copy(ei_hbm.at[1, pl.ds(base, b3)], didx_v[k],
                             isem[k])

        def fetch_rows(k):
            pltpu.async_copy(y_sh.at[sidx_v[k]], rows_v[k], rsem[k])
            pltpu.async_copy(dinv_sh.at[didx_v[k]], dval_v[k], dsem[k])

        def wait_idx(i, k):
            base = c * e2 + s * ept + i * b3
            pltpu.make_async_copy(ei_hbm.at[0, pl.ds(base, b3)], sidx_v[k],
                                  isem[k]).wait()
            pltpu.make_async_copy(ei_hbm.at[1, pl.ds(base, b3)], didx_v[k],
                                  isem[k]).wait()

        fetch_idx(0, 0)
        wait_idx(0, 0)
        fetch_rows(0)
        fetch_idx(1, 1)

        def step(j, carry):
            for k in (0, 1):
                i = j * 2 + k
                kn = 1 - k
                pltpu.make_async_copy(y_sh.at[sidx_v[k]], rows_v[k],
                                      rsem[k]).wait()
                pltpu.make_async_copy(dinv_sh.at[didx_v[k]], dval_v[k],
                                      dsem[k]).wait()

                @pl.when(i + 1 < nit)
                def _():
                    wait_idx(i + 1, kn)
                    fetch_rows(kn)

                pltpu.sync_copy(rows_v[k], agg_sh.at[didx_v[k]], add=True)
                pltpu.sync_copy(dval_v[k], t_sh.at[sidx_v[k]], add=True)

                @pl.when(i + 2 < nit)
                def _():
                    fetch_idx(i + 2, k)
            return carry

        lax.fori_loop(0, nit // 2, step, 0)
        plsc.subcore_barrier()

        def phased(j, carry):
            a = a0 + j * chunk
            i0 = pltpu.async_copy(agg_sh.at[pl.ds(a, chunk), :], xbuf,
                                  isem[0])
            i1 = pltpu.async_copy(dinv_sh.at[pl.ds(a, chunk)], dvbuf,
                                  isem[1])
            i2 = pltpu.async_copy(t_sh.at[pl.ds(a, chunk)], tbuf,
                                  rsem[0])
            i0.wait()
            i1.wait()
            i2.wait()
            for v in range(chunk * f // LANES):
                fi = v * LANES + iota
                ri = lax.shift_right_logical(fi, 3)
                ci = lax.bitwise_and(fi, 7)
                av = plsc.load_gather(xbuf, [ri, ci])
                dv = plsc.load_gather(dvbuf, [ri])
                ro = lax.shift_right_logical(fi, 7)
                co = lax.bitwise_and(fi, 127)
                plsc.store_scatter(obuf, [ro, co], av * dv)
            for v in range(chunk // LANES):
                sl = pl.ds(v * LANES, LANES)
                wbuf[sl] = dvbuf[sl] * tbuf[sl]
            o0 = pltpu.async_copy(
                obuf, a_out.at[c, pl.ds(a * f // 128, rows_per_chunk), :],
                isem[0])
            o1 = pltpu.async_copy(wbuf, w_out.at[c, pl.ds(a, chunk)],
                                  isem[1])
            o0.wait()
            o1.wait()
            return carry

        lax.fori_loop(0, nchunks, phased, 0)

        if npad > n:
            @pl.when(s == NS - 1)
            def _():
                pltpu.sync_copy(zbuf1.at[pl.ds(0, npad - n)],
                                w_out.at[c, pl.ds(n, npad - n)])

    return agg_kernel


def _make_dense(n: int, f: int):
    def dense_body(a_in, wpt, mask, fsel, w1bd, b1t, st,
                   w2r, b2r, wsr, bsr, wor, bor, out):
        a = a_in[0] + a_in[1]
        p = jnp.dot(a, w1bd[...],
                    preferred_element_type=jnp.float32)
        h = jnp.maximum(p + b1t[...], 0.0)
        cm = jnp.dot(wpt[...], h,
                     preferred_element_type=jnp.float32)
        colsum = jnp.sum(cm * mask[...], axis=0, keepdims=True)
        colsum = jnp.dot(colsum, fsel[...],
                         preferred_element_type=jnp.float32)
        gm = jnp.dot(colsum * (1.0 / n), w2r[...],
                     preferred_element_type=jnp.float32) + b2r[...]
        se = jnp.maximum(
            jnp.dot(st[...], wsr[...],
                    preferred_element_type=jnp.float32) + bsr[...], 0.0)
        z = jnp.concatenate([gm, se], axis=1)
        out[...] = jnp.dot(z, wor[...],
                           preferred_element_type=jnp.float32) + bor[...]

    return pl.pallas_call(
        dense_body,
        out_shape=jax.ShapeDtypeStruct((1, 2), jnp.float32),
    )


def kernel(x, edge_index, state, W1, b1, W2, b2, Ws, bs, Wo, bo):
    n, f = x.shape
    e = edge_index.shape[1]
    g = 128 // f
    npad = ((n + 127) // 128) * 128

    degp = _make_deg(n, e)(edge_index)
    a_out, w_out = _make_agg(n, e, f)(edge_index, x, degp)

    w1bd = (jnp.eye(g, dtype=jnp.float32)[:, None, :, None]
            * W1[None, :, None, :]).reshape(f * g, 64 * g)
    b1t = jnp.tile(b1, g)[None]

    nrow = n * f // 128
    wsum = (w_out[0] + w_out[1])[:n]
    wpt = wsum.reshape(nrow, g).T
    mask = jnp.repeat(jnp.eye(g, dtype=jnp.float32), 64, axis=1)
    fsel = jnp.tile(jnp.eye(64, dtype=jnp.float32), (g, 1))

    out = _make_dense(n, f)(
        a_out, wpt, mask, fsel, w1bd, b1t, state,
        W2, b2[None], Ws, bs[None], Wo, bo[None])
    return out


# speedup vs baseline: 1.2368x; 1.1041x over previous
import functools

import jax
import jax.numpy as jnp
from jax import lax
from jax.experimental import pallas as pl
from jax.experimental.pallas import tpu as pltpu
from jax.experimental.pallas import tpu_sc as plsc

NC = 2
NS = 16
NW = NC * NS
LANES = 16
MAGIC = 0x5F3759DF


def _pick_block(ept: int, maxb: int, even_count: bool = False) -> int:
    align = 8 if even_count else 16
    for b in range(min(maxb, ept), 15, -1):
        if ept % b == 0 and b % align == 0:
            if even_count and (ept // b) % 2 != 0:
                continue
            return b
    return 0


def _iota16():
    return lax.broadcasted_iota(jnp.int32, (LANES,), 0)


def _make_deg(n: int, e: int):
    ept = e // NW
    b1 = _pick_block(ept, 10000)
    e2 = e // NC
    mesh = plsc.VectorSubcoreMesh(core_axis_name="c", subcore_axis_name="s")

    @functools.partial(
        pl.kernel,
        out_type=jax.ShapeDtypeStruct((NC, n), jnp.float32),
        mesh=mesh,
        scratch_types=[
            [pltpu.VMEM((b1,), jnp.int32)] * 2,
            pltpu.VMEM((b1,), jnp.float32),
            pltpu.VMEM((b1,), jnp.float32),
            pltpu.VMEM_SHARED((n,), jnp.float32),
            [pltpu.SemaphoreType.DMA] * 2,
        ],
        compiler_params=pltpu.CompilerParams(use_tc_tiling_on_sc=False, needs_layout_passes=False),
    )
    def deg_kernel(ei_hbm, outp, idx_v, ones_v, zero_v, deg_sh, isem):
        c = lax.axis_index("c")
        s = lax.axis_index("s")

        nit = ept // b1

        def fetch(i):
            base = c * e2 + s * ept + i * b1
            pltpu.async_copy(ei_hbm.at[1, pl.ds(base, b1)], idx_v[i % 2],
                             isem[i % 2])

        def wait(i):
            base = c * e2 + s * ept + i * b1
            pltpu.make_async_copy(ei_hbm.at[1, pl.ds(base, b1)],
                                  idx_v[i % 2], isem[i % 2]).wait()

        fetch(0)

        def fill(i, carry):
            ones_v[pl.ds(i * 16, 16)] = jnp.full((16,), 1.0, jnp.float32)
            zero_v[pl.ds(i * 16, 16)] = jnp.zeros((16,), jnp.float32)
            return carry

        lax.fori_loop(0, b1 // 16, fill, 0)

        @pl.when(s == 0)
        def _():
            def zchunk(i, carry):
                pltpu.sync_copy(zero_v, deg_sh.at[pl.ds(i * b1, b1)])
                return carry
            lax.fori_loop(0, n // b1, zchunk, 0)
            if n % b1:
                pltpu.sync_copy(zero_v.at[pl.ds(0, n % b1)],
                                deg_sh.at[pl.ds((n // b1) * b1, n % b1)])

        plsc.subcore_barrier()

        for i in range(nit):
            wait(i)
            if i + 1 < nit:
                fetch(i + 1)
            pltpu.sync_copy(ones_v, deg_sh.at[idx_v[i % 2]], add=True)
        plsc.subcore_barrier()

        @pl.when(s == 0)
        def _():
            pltpu.sync_copy(deg_sh, outp.at[c])

    return deg_kernel


def _make_agg(n: int, e: int, f: int):
    ept = e // NW
    b3 = _pick_block(ept, 2000, even_count=True)
    e2 = e // NC
    nit = ept // b3
    chunk = 400
    big = ((n // NS + chunk - 1) // chunk) * chunk
    rest = n - big * (NS - 1)
    assert rest > 0 and rest % chunk == 0 and big % chunk == 0
    nrow = n * f // 128
    npad = ((n + 127) // 128) * 128
    rows_per_chunk = chunk * f // 128
    mesh = plsc.VectorSubcoreMesh(core_axis_name="c", subcore_axis_name="s")

    @functools.partial(
        pl.kernel,
        out_type=(
            jax.ShapeDtypeStruct((NC, nrow, 128), jnp.float32),
            jax.ShapeDtypeStruct((NC, npad), jnp.float32),
            jax.ShapeDtypeStruct((n, f), jnp.float32),
        ),
        mesh=mesh,
        scratch_types=[
            [pltpu.VMEM((b3,), jnp.int32)] * 2,
            [pltpu.VMEM((b3,), jnp.int32)] * 2,
            [pltpu.VMEM((b3, f), jnp.float32)] * 2,
            [pltpu.VMEM((b3,), jnp.float32)] * 2,
            pltpu.VMEM((chunk, f), jnp.float32),
            pltpu.VMEM((chunk, f), jnp.float32),
            pltpu.VMEM((chunk,), jnp.float32),
            pltpu.VMEM((chunk,), jnp.float32),
            pltpu.VMEM((chunk,), jnp.float32),
            pltpu.VMEM((chunk,), jnp.float32),
            pltpu.VMEM((chunk,), jnp.float32),
            pltpu.VMEM((rows_per_chunk, 128), jnp.float32),
            pltpu.VMEM((chunk, f), jnp.float32),
            pltpu.VMEM((chunk,), jnp.float32),
            pltpu.VMEM_SHARED((n,), jnp.float32),
            pltpu.VMEM_SHARED((n, f), jnp.float32),
            pltpu.VMEM_SHARED((n,), jnp.float32),
            [pltpu.SemaphoreType.DMA] * 2,
            [pltpu.SemaphoreType.DMA] * 2,
            [pltpu.SemaphoreType.DMA] * 2,
        ],
        compiler_params=pltpu.CompilerParams(use_tc_tiling_on_sc=False, needs_layout_passes=False),
    )
    def agg_kernel(ei_hbm, x_hbm, degp_hbm, a_out, w_out, y_hbm,
                   sidx_v, didx_v, rows_v, dval_v,
                   xbuf, ybuf, dbuf0, dbuf1, dvbuf, tbuf, wbuf, obuf,
                   zbuf8, zbuf1, dinv_sh, agg_sh, t_sh,
                   isem, rsem, dsem):
        c = lax.axis_index("c")
        s = lax.axis_index("s")
        iota = _iota16()
        a0 = s * big
        nchunks = jnp.where(s == NS - 1, rest // chunk, big // chunk)

        zf32 = jnp.zeros((LANES,), jnp.float32)
        for v in range(chunk * f // LANES):
            fi = v * LANES + iota
            plsc.store_scatter(
                zbuf8, [lax.shift_right_logical(fi, 3),
                        lax.bitwise_and(fi, 7)], zf32)
        for v in range(chunk // LANES):
            zbuf1[pl.ds(v * LANES, LANES)] = zf32

        def phaseb(j, carry):
            a = a0 + j * chunk
            d0 = pltpu.async_copy(degp_hbm.at[0, pl.ds(a, chunk)], dbuf0,
                                  isem[0])
            d1 = pltpu.async_copy(degp_hbm.at[1, pl.ds(a, chunk)], dbuf1,
                                  isem[1])
            dx = pltpu.async_copy(x_hbm.at[pl.ds(a, chunk), :], xbuf,
                                  rsem[0])
            d0.wait()
            d1.wait()
            dx.wait()
            for v in range(chunk // LANES):
                sl = pl.ds(v * LANES, LANES)
                d0 = dbuf0[sl] + dbuf1[sl] + 1.0
                i0 = plsc.bitcast(d0, jnp.int32)
                i1 = MAGIC - lax.shift_right_arithmetic(i0, 1)
                r = plsc.bitcast(i1, jnp.float32)
                for _ in range(3):
                    r = r * (1.5 - 0.5 * d0 * r * r)
                dvbuf[sl] = r
            for v in range(chunk * f // LANES):
                fi = v * LANES + iota
                ri = lax.shift_right_logical(fi, 3)
                ci = lax.bitwise_and(fi, 7)
                xv = plsc.load_gather(xbuf, [ri, ci])
                dv = plsc.load_gather(dvbuf, [ri])
                plsc.store_scatter(ybuf, [ri, ci], xv * dv)
            o0 = pltpu.async_copy(dvbuf, dinv_sh.at[pl.ds(a, chunk)],
                                  isem[0])
            o1 = pltpu.async_copy(ybuf, y_hbm.at[pl.ds(a, chunk), :],
                                  isem[1])

            @pl.when(c == 0)
            def _():
                pltpu.async_copy(ybuf, agg_sh.at[pl.ds(a, chunk), :],
                                 rsem[0]).wait()
                pltpu.async_copy(dvbuf, t_sh.at[pl.ds(a, chunk)],
                                 rsem[1]).wait()

            @pl.when(c != 0)
            def _():
                pltpu.async_copy(zbuf8, agg_sh.at[pl.ds(a, chunk), :],
                                 rsem[0]).wait()
                pltpu.async_copy(zbuf1, t_sh.at[pl.ds(a, chunk)],
                                 rsem[1]).wait()
            o0.wait()
            o1.wait()
            return carry

        lax.fori_loop(0, nchunks, phaseb, 0)
        plsc.subcore_barrier()

        def fetch_idx(i, k):
            base = c * e2 + s * ept + i * b3
            pltpu.async_copy(ei_hbm.at[0, pl.ds(base, b3)], sidx_v[k],
                             isem[k])
            pltpu.async_copy(ei_hbm.at[1, pl.ds(base, b3)], didx_v[k],
                             isem[k])

        def fetch_rows(k):
            pltpu.async_copy(y_hbm.at[sidx_v[k]], rows_v[k], rsem[k])
            pltpu.async_copy(dinv_sh.at[didx_v[k]], dval_v[k], dsem[k])

        def wait_idx(i, k):
            base = c * e2 + s * ept + i * b3
            pltpu.make_async_copy(ei_hbm.at[0, pl.ds(base, b3)], sidx_v[k],
                                  isem[k]).wait()
            pltpu.make_async_copy(ei_hbm.at[1, pl.ds(base, b3)], didx_v[k],
                                  isem[k]).wait()

        fetch_idx(0, 0)
        wait_idx(0, 0)
        fetch_rows(0)
        fetch_idx(1, 1)

        def step(j, carry):
            for k in (0, 1):
                i = j * 2 + k
                kn = 1 - k
                pltpu.make_async_copy(dinv_sh.at[didx_v[k]], dval_v[k],
                                      dsem[k]).wait()
                pltpu.sync_copy(dval_v[k], t_sh.at[sidx_v[k]], add=True)

                @pl.when(i + 1 < nit)
                def _():
                    wait_idx(i + 1, kn)
                    fetch_rows(kn)

                pltpu.make_async_copy(y_hbm.at[sidx_v[k]], rows_v[k],
                                      rsem[k]).wait()
                pltpu.sync_copy(rows_v[k], agg_sh.at[didx_v[k]], add=True)

                @pl.when(i + 2 < nit)
                def _():
                    fetch_idx(i + 2, k)
            return carry

        lax.fori_loop(0, nit // 2, step, 0)
        plsc.subcore_barrier()

        def phased(j, carry):
            a = a0 + j * chunk
            i0 = pltpu.async_copy(agg_sh.at[pl.ds(a, chunk), :], xbuf,
                                  isem[0])
            i1 = pltpu.async_copy(dinv_sh.at[pl.ds(a, chunk)], dvbuf,
                                  isem[1])
            i2 = pltpu.async_copy(t_sh.at[pl.ds(a, chunk)], tbuf,
                                  rsem[0])
            i0.wait()
            i1.wait()
            i2.wait()
            for v in range(chunk * f // LANES):
                fi = v * LANES + iota
                ri = lax.shift_right_logical(fi, 3)
                ci = lax.bitwise_and(fi, 7)
                av = plsc.load_gather(xbuf, [ri, ci])
                dv = plsc.load_gather(dvbuf, [ri])
                ro = lax.shift_right_logical(fi, 7)
                co = lax.bitwise_and(fi, 127)
                plsc.store_scatter(obuf, [ro, co], av * dv)
            for v in range(chunk // LANES):
                sl = pl.ds(v * LANES, LANES)
                wbuf[sl] = dvbuf[sl] * tbuf[sl]
            o0 = pltpu.async_copy(
                obuf, a_out.at[c, pl.ds(a * f // 128, rows_per_chunk), :],
                isem[0])
            o1 = pltpu.async_copy(wbuf, w_out.at[c, pl.ds(a, chunk)],
                                  isem[1])
            o0.wait()
            o1.wait()
            return carry

        lax.fori_loop(0, nchunks, phased, 0)

        if npad > n:
            @pl.when(s == NS - 1)
            def _():
                pltpu.sync_copy(zbuf1.at[pl.ds(0, npad - n)],
                                w_out.at[c, pl.ds(n, npad - n)])

    return agg_kernel


def _make_dense(n: int, f: int):
    def dense_body(a_in, wpt, mask, fsel, w1bd, b1t, st,
                   w2r, b2r, wsr, bsr, wor, bor, out):
        a = a_in[0] + a_in[1]
        p = jnp.dot(a, w1bd[...],
                    preferred_element_type=jnp.float32)
        h = jnp.maximum(p + b1t[...], 0.0)
        cm = jnp.dot(wpt[...], h,
                     preferred_element_type=jnp.float32)
        colsum = jnp.sum(cm * mask[...], axis=0, keepdims=True)
        colsum = jnp.dot(colsum, fsel[...],
                         preferred_element_type=jnp.float32)
        gm = jnp.dot(colsum * (1.0 / n), w2r[...],
                     preferred_element_type=jnp.float32) + b2r[...]
        se = jnp.maximum(
            jnp.dot(st[...], wsr[...],
                    preferred_element_type=jnp.float32) + bsr[...], 0.0)
        z = jnp.concatenate([gm, se], axis=1)
        out[...] = jnp.dot(z, wor[...],
                           preferred_element_type=jnp.float32) + bor[...]

    return pl.pallas_call(
        dense_body,
        out_shape=jax.ShapeDtypeStruct((1, 2), jnp.float32),
    )


def kernel(x, edge_index, state, W1, b1, W2, b2, Ws, bs, Wo, bo):
    n, f = x.shape
    e = edge_index.shape[1]
    g = 128 // f
    npad = ((n + 127) // 128) * 128

    degp = _make_deg(n, e)(edge_index)
    a_out, w_out, _y_unused = _make_agg(n, e, f)(edge_index, x, degp)

    w1bd = (jnp.eye(g, dtype=jnp.float32)[:, None, :, None]
            * W1[None, :, None, :]).reshape(f * g, 64 * g)
    b1t = jnp.tile(b1, g)[None]

    nrow = n * f // 128
    wsum = (w_out[0] + w_out[1])[:n]
    wpt = wsum.reshape(nrow, g).T
    mask = jnp.repeat(jnp.eye(g, dtype=jnp.float32), 64, axis=1)
    fsel = jnp.tile(jnp.eye(64, dtype=jnp.float32), (g, 1))

    out = _make_dense(n, f)(
        a_out, wpt, mask, fsel, w1bd, b1t, state,
        W2, b2[None], Ws, bs[None], Wo, bo[None])
    return out


# speedup vs baseline: 1.2867x; 1.0403x over previous
import functools

import jax
import jax.numpy as jnp
from jax import lax
from jax.experimental import pallas as pl
from jax.experimental.pallas import tpu as pltpu
from jax.experimental.pallas import tpu_sc as plsc

NC = 2
NS = 16
NW = NC * NS
LANES = 16
MAGIC = 0x5F3759DF


def _pick_block(ept: int, maxb: int, even_count: bool = False) -> int:
    align = 8 if even_count else 16
    for b in range(min(maxb, ept), 15, -1):
        if ept % b == 0 and b % align == 0:
            if even_count and (ept // b) % 2 != 0:
                continue
            return b
    return 0


def _iota16():
    return lax.broadcasted_iota(jnp.int32, (LANES,), 0)


def _make_deg(n: int, e: int):
    ept = e // NW
    b1 = _pick_block(ept, 10000)
    e2 = e // NC
    mesh = plsc.VectorSubcoreMesh(core_axis_name="c", subcore_axis_name="s")

    @functools.partial(
        pl.kernel,
        out_type=jax.ShapeDtypeStruct((NC, n), jnp.float32),
        mesh=mesh,
        scratch_types=[
            [pltpu.VMEM((b1,), jnp.int32)] * 2,
            pltpu.VMEM((b1,), jnp.float32),
            pltpu.VMEM((b1,), jnp.float32),
            pltpu.VMEM_SHARED((n,), jnp.float32),
            [pltpu.SemaphoreType.DMA] * 2,
        ],
        compiler_params=pltpu.CompilerParams(use_tc_tiling_on_sc=False, needs_layout_passes=False),
    )
    def deg_kernel(ei_hbm, outp, idx_v, ones_v, zero_v, deg_sh, isem):
        c = lax.axis_index("c")
        s = lax.axis_index("s")

        nit = ept // b1

        def fetch(i):
            base = c * e2 + s * ept + i * b1
            pltpu.async_copy(ei_hbm.at[1, pl.ds(base, b1)], idx_v[i % 2],
                             isem[i % 2])

        def wait(i):
            base = c * e2 + s * ept + i * b1
            pltpu.make_async_copy(ei_hbm.at[1, pl.ds(base, b1)],
                                  idx_v[i % 2], isem[i % 2]).wait()

        fetch(0)

        def fill(i, carry):
            ones_v[pl.ds(i * 16, 16)] = jnp.full((16,), 1.0, jnp.float32)
            zero_v[pl.ds(i * 16, 16)] = jnp.zeros((16,), jnp.float32)
            return carry

        lax.fori_loop(0, b1 // 16, fill, 0)

        @pl.when(s == 0)
        def _():
            def zchunk(i, carry):
                pltpu.sync_copy(zero_v, deg_sh.at[pl.ds(i * b1, b1)])
                return carry
            lax.fori_loop(0, n // b1, zchunk, 0)
            if n % b1:
                pltpu.sync_copy(zero_v.at[pl.ds(0, n % b1)],
                                deg_sh.at[pl.ds((n // b1) * b1, n % b1)])

        plsc.subcore_barrier()

        for i in range(nit):
            wait(i)
            if i + 1 < nit:
                fetch(i + 1)
            pltpu.sync_copy(ones_v, deg_sh.at[idx_v[i % 2]], add=True)
        plsc.subcore_barrier()

        @pl.when(s == 0)
        def _():
            pltpu.sync_copy(deg_sh, outp.at[c])

    return deg_kernel


def _make_agg(n: int, e: int, f: int):
    ept = e // NW
    b3 = _pick_block(ept, 2000)
    e2 = e // NC
    nit = ept // b3
    chunk = 400
    big = ((n // NS + chunk - 1) // chunk) * chunk
    rest = n - big * (NS - 1)
    assert rest > 0 and rest % chunk == 0 and big % chunk == 0
    nrow = n * f // 128
    npad = ((n + 127) // 128) * 128
    rows_per_chunk = chunk * f // 128
    mesh = plsc.VectorSubcoreMesh(core_axis_name="c", subcore_axis_name="s")

    @functools.partial(
        pl.kernel,
        out_type=(
            jax.ShapeDtypeStruct((NC, nrow, 128), jnp.float32),
            jax.ShapeDtypeStruct((NC, npad), jnp.float32),
            jax.ShapeDtypeStruct((n, f), jnp.float32),
        ),
        mesh=mesh,
        scratch_types=[
            [pltpu.VMEM((b3,), jnp.int32)] * 2,
            [pltpu.VMEM((b3,), jnp.int32)] * 2,
            [pltpu.VMEM((b3, f), jnp.float32)] * 2,
            [pltpu.VMEM((b3,), jnp.float32)] * 2,
            pltpu.VMEM((chunk, f), jnp.float32),
            pltpu.VMEM((chunk, f), jnp.float32),
            pltpu.VMEM((chunk,), jnp.float32),
            pltpu.VMEM((chunk,), jnp.float32),
            pltpu.VMEM((chunk,), jnp.float32),
            pltpu.VMEM((chunk,), jnp.float32),
            pltpu.VMEM((chunk,), jnp.float32),
            pltpu.VMEM((rows_per_chunk, 128), jnp.float32),
            pltpu.VMEM((chunk, f), jnp.float32),
            pltpu.VMEM((chunk,), jnp.float32),
            pltpu.VMEM_SHARED((n,), jnp.float32),
            pltpu.VMEM_SHARED((n, f), jnp.float32),
            pltpu.VMEM_SHARED((n,), jnp.float32),
            [pltpu.SemaphoreType.DMA] * 2,
            [pltpu.SemaphoreType.DMA] * 2,
            [pltpu.SemaphoreType.DMA] * 2,
        ],
        compiler_params=pltpu.CompilerParams(use_tc_tiling_on_sc=False, needs_layout_passes=False),
    )
    def agg_kernel(ei_hbm, x_hbm, degp_hbm, a_out, w_out, y_hbm,
                   sidx_v, didx_v, rows_v, dval_v,
                   xbuf, ybuf, dbuf0, dbuf1, dvbuf, tbuf, wbuf, obuf,
                   zbuf8, zbuf1, dinv_sh, agg_sh, t_sh,
                   isem, rsem, dsem):
        c = lax.axis_index("c")
        s = lax.axis_index("s")
        iota = _iota16()
        a0 = s * big
        nchunks = jnp.where(s == NS - 1, rest // chunk, big // chunk)

        zf32 = jnp.zeros((LANES,), jnp.float32)
        for v in range(chunk * f // LANES):
            fi = v * LANES + iota
            plsc.store_scatter(
                zbuf8, [lax.shift_right_logical(fi, 3),
                        lax.bitwise_and(fi, 7)], zf32)
        for v in range(chunk // LANES):
            zbuf1[pl.ds(v * LANES, LANES)] = zf32

        def phaseb(j, carry):
            a = a0 + j * chunk
            d0 = pltpu.async_copy(degp_hbm.at[0, pl.ds(a, chunk)], dbuf0,
                                  isem[0])
            d1 = pltpu.async_copy(degp_hbm.at[1, pl.ds(a, chunk)], dbuf1,
                                  isem[1])
            dx = pltpu.async_copy(x_hbm.at[pl.ds(a, chunk), :], xbuf,
                                  rsem[0])
            d0.wait()
            d1.wait()
            dx.wait()
            for v in range(chunk // LANES):
                sl = pl.ds(v * LANES, LANES)
                d0 = dbuf0[sl] + dbuf1[sl] + 1.0
                i0 = plsc.bitcast(d0, jnp.int32)
                i1 = MAGIC - lax.shift_right_arithmetic(i0, 1)
                r = plsc.bitcast(i1, jnp.float32)
                for _ in range(3):
                    r = r * (1.5 - 0.5 * d0 * r * r)
                dvbuf[sl] = r
            for v in range(chunk * f // LANES):
                fi = v * LANES + iota
                ri = lax.shift_right_logical(fi, 3)
                ci = lax.bitwise_and(fi, 7)
                xv = plsc.load_gather(xbuf, [ri, ci])
                dv = plsc.load_gather(dvbuf, [ri])
                plsc.store_scatter(ybuf, [ri, ci], xv * dv)
            o0 = pltpu.async_copy(dvbuf, dinv_sh.at[pl.ds(a, chunk)],
                                  isem[0])
            o1 = pltpu.async_copy(ybuf, y_hbm.at[pl.ds(a, chunk), :],
                                  isem[1])

            @pl.when(c == 0)
            def _():
                pltpu.async_copy(ybuf, agg_sh.at[pl.ds(a, chunk), :],
                                 rsem[0]).wait()
                pltpu.async_copy(dvbuf, t_sh.at[pl.ds(a, chunk)],
                                 rsem[1]).wait()

            @pl.when(c != 0)
            def _():
                pltpu.async_copy(zbuf8, agg_sh.at[pl.ds(a, chunk), :],
                                 rsem[0]).wait()
                pltpu.async_copy(zbuf1, t_sh.at[pl.ds(a, chunk)],
                                 rsem[1]).wait()
            o0.wait()
            o1.wait()
            return carry

        lax.fori_loop(0, nchunks, phaseb, 0)
        plsc.subcore_barrier()

        def fetch_idx(i, k):
            base = c * e2 + s * ept + i * b3
            pltpu.async_copy(ei_hbm.at[0, pl.ds(base, b3)], sidx_v[k],
                             isem[k])
            pltpu.async_copy(ei_hbm.at[1, pl.ds(base, b3)], didx_v[k],
                             isem[k])

        def fetch_rows(k):
            pltpu.async_copy(y_hbm.at[sidx_v[k]], rows_v[k], rsem[k])
            pltpu.async_copy(dinv_sh.at[didx_v[k]], dval_v[k], dsem[k])

        def wait_idx(i, k):
            base = c * e2 + s * ept + i * b3
            pltpu.make_async_copy(ei_hbm.at[0, pl.ds(base, b3)], sidx_v[k],
                                  isem[k]).wait()
            pltpu.make_async_copy(ei_hbm.at[1, pl.ds(base, b3)], didx_v[k],
                                  isem[k]).wait()

        fetch_idx(0, 0)
        wait_idx(0, 0)
        fetch_rows(0)
        fetch_idx(1, 1)

        def edge_body(i, k):
            kn = 1 - k
            pltpu.make_async_copy(dinv_sh.at[didx_v[k]], dval_v[k],
                                  dsem[k]).wait()
            pltpu.sync_copy(dval_v[k], t_sh.at[sidx_v[k]], add=True)

            @pl.when(i + 1 < nit)
            def _():
                wait_idx(i + 1, kn)
                fetch_rows(kn)

            pltpu.make_async_copy(y_hbm.at[sidx_v[k]], rows_v[k],
                                  rsem[k]).wait()
            pltpu.sync_copy(rows_v[k], agg_sh.at[didx_v[k]], add=True)

            @pl.when(i + 2 < nit)
            def _():
                fetch_idx(i + 2, k)

        if nit % 2:
            edge_body(0, 0)

            def step(j, carry):
                for q in (1, 2):
                    i = j * 2 + q
                    edge_body(i, (q % 2) ^ (0 if nit % 2 else 1))
                return carry
        else:
            def step(j, carry):
                for q in (0, 1):
                    edge_body(j * 2 + q, q)
                return carry

        lax.fori_loop(0, nit // 2, step, 0)
        plsc.subcore_barrier()

        def phased(j, carry):
            a = a0 + j * chunk
            i0 = pltpu.async_copy(agg_sh.at[pl.ds(a, chunk), :], xbuf,
                                  isem[0])
            i1 = pltpu.async_copy(dinv_sh.at[pl.ds(a, chunk)], dvbuf,
                                  isem[1])
            i2 = pltpu.async_copy(t_sh.at[pl.ds(a, chunk)], tbuf,
                                  rsem[0])
            i0.wait()
            i1.wait()
            i2.wait()
            for v in range(chunk * f // LANES):
                fi = v * LANES + iota
                ri = lax.shift_right_logical(fi, 3)
                ci = lax.bitwise_and(fi, 7)
                av = plsc.load_gather(xbuf, [ri, ci])
                dv = plsc.load_gather(dvbuf, [ri])
                ro = lax.shift_right_logical(fi, 7)
                co = lax.bitwise_and(fi, 127)
                plsc.store_scatter(obuf, [ro, co], av * dv)
            for v in range(chunk // LANES):
                sl = pl.ds(v * LANES, LANES)
                wbuf[sl] = dvbuf[sl] * tbuf[sl]
            o0 = pltpu.async_copy(
                obuf, a_out.at[c, pl.ds(a * f // 128, rows_per_chunk), :],
                isem[0])
            o1 = pltpu.async_copy(wbuf, w_out.at[c, pl.ds(a, chunk)],
                                  isem[1])
            o0.wait()
            o1.wait()
            return carry

        lax.fori_loop(0, nchunks, phased, 0)

        if npad > n:
            @pl.when(s == NS - 1)
            def _():
                pltpu.sync_copy(zbuf1.at[pl.ds(0, npad - n)],
                                w_out.at[c, pl.ds(n, npad - n)])

    return agg_kernel


def _make_dense(n: int, f: int):
    def dense_body(a_in, wpt, mask, fsel, w1bd, b1t, st,
                   w2r, b2r, wsr, bsr, wor, bor, out):
        a = a_in[0] + a_in[1]
        p = jnp.dot(a, w1bd[...],
                    preferred_element_type=jnp.float32)
        h = jnp.maximum(p + b1t[...], 0.0)
        cm = jnp.dot(wpt[...], h,
                     preferred_element_type=jnp.float32)
        colsum = jnp.sum(cm * mask[...], axis=0, keepdims=True)
        colsum = jnp.dot(colsum, fsel[...],
                         preferred_element_type=jnp.float32)
        gm = jnp.dot(colsum * (1.0 / n), w2r[...],
                     preferred_element_type=jnp.float32) + b2r[...]
        se = jnp.maximum(
            jnp.dot(st[...], wsr[...],
                    preferred_element_type=jnp.float32) + bsr[...], 0.0)
        z = jnp.concatenate([gm, se], axis=1)
        out[...] = jnp.dot(z, wor[...],
                           preferred_element_type=jnp.float32) + bor[...]

    return pl.pallas_call(
        dense_body,
        out_shape=jax.ShapeDtypeStruct((1, 2), jnp.float32),
    )


def kernel(x, edge_index, state, W1, b1, W2, b2, Ws, bs, Wo, bo):
    n, f = x.shape
    e = edge_index.shape[1]
    g = 128 // f
    npad = ((n + 127) // 128) * 128

    degp = _make_deg(n, e)(edge_index)
    a_out, w_out, _y_unused = _make_agg(n, e, f)(edge_index, x, degp)

    w1bd = (jnp.eye(g, dtype=jnp.float32)[:, None, :, None]
            * W1[None, :, None, :]).reshape(f * g, 64 * g)
    b1t = jnp.tile(b1, g)[None]

    nrow = n * f // 128
    wsum = (w_out[0] + w_out[1])[:n]
    wpt = wsum.reshape(nrow, g).T
    mask = jnp.repeat(jnp.eye(g, dtype=jnp.float32), 64, axis=1)
    fsel = jnp.tile(jnp.eye(64, dtype=jnp.float32), (g, 1))

    out = _make_dense(n, f)(
        a_out, wpt, mask, fsel, w1bd, b1t, state,
        W2, b2[None], Ws, bs[None], Wo, bo[None])
    return out
